# baseline (speedup 1.0000x reference)
import functools

import jax
import jax.numpy as jnp
from jax import lax
from jax.experimental import pallas as pl
from jax.experimental.pallas import tpu as pltpu
from jax.experimental.pallas import tpu_sc as plsc


def _sc_copy(w_hbm, o_hbm, buf, in_sems, out_sems, *, rows_w, chunk,
             n_chunks, nbuf):
    wid = lax.axis_index("s") * 2 + lax.axis_index("c")
    base = wid * rows_w
    for c in range(n_chunks):
        b = c % nbuf
        if c >= nbuf:
            pltpu.make_async_copy(
                buf.at[b],
                o_hbm.at[pl.ds(base + (c - nbuf) * chunk, chunk), :],
                out_sems.at[b],
            ).wait()
        in_copy = pltpu.make_async_copy(
            w_hbm.at[pl.ds(base + c * chunk, chunk), :],
            buf.at[b],
            in_sems.at[b],
        )
        in_copy.start()
        in_copy.wait()
        pltpu.make_async_copy(
            buf.at[b],
            o_hbm.at[pl.ds(base + c * chunk, chunk), :],
            out_sems.at[b],
        ).start()
    for c in range(max(n_chunks - nbuf, 0), n_chunks):
        b = c % nbuf
        pltpu.make_async_copy(
            buf.at[b],
            o_hbm.at[pl.ds(base + c * chunk, chunk), :],
            out_sems.at[b],
        ).wait()


def kernel(x, W):
    del x
    S, D = W.shape
    n_workers = 32
    rows_w = S // n_workers
    chunk = 64
    n_chunks = rows_w // chunk
    nbuf = 2
    mesh = plsc.VectorSubcoreMesh(core_axis_name="c", subcore_axis_name="s")
    body = functools.partial(
        _sc_copy, rows_w=rows_w, chunk=chunk, n_chunks=n_chunks, nbuf=nbuf)
    k = pl.kernel(
        body,
        out_type=jax.ShapeDtypeStruct((S, D), W.dtype),
        mesh=mesh,
        scratch_types=[
            pltpu.VMEM((nbuf, chunk, D), W.dtype),
            pltpu.SemaphoreType.DMA((nbuf,)),
            pltpu.SemaphoreType.DMA((nbuf,)),
        ],
    )
    return k(W)


# speedup vs baseline: 1.0611x; 1.0611x over previous
import functools

import jax
import jax.numpy as jnp
from jax import lax
from jax.experimental import pallas as pl
from jax.experimental.pallas import tpu as pltpu
from jax.experimental.pallas import tpu_sc as plsc


def _sc_copy(w_hbm, o_hbm, buf, in_sems, out_sems, *, rows_w, chunk,
             n_chunks, nbuf):
    wid = lax.axis_index("s") * 2 + lax.axis_index("c")
    base = wid * rows_w

    def in_copy(c):
        return pltpu.make_async_copy(
            w_hbm.at[pl.ds(base + c * chunk, chunk), :],
            buf.at[c % nbuf],
            in_sems.at[c % nbuf],
        )

    def out_copy(c):
        return pltpu.make_async_copy(
            buf.at[c % nbuf],
            o_hbm.at[pl.ds(base + c * chunk, chunk), :],
            out_sems.at[c % nbuf],
        )

    for c in range(min(nbuf, n_chunks)):
        in_copy(c).start()
    for c in range(n_chunks):
        in_copy(c).wait()
        out_copy(c).start()
        nxt = c + nbuf
        if nxt < n_chunks:
            out_copy(c).wait()
            in_copy(nxt).start()
    for c in range(max(n_chunks - nbuf, 0), n_chunks):
        out_copy(c).wait()


def kernel(x, W):
    del x
    S, D = W.shape
    n_workers = 32
    rows_w = S // n_workers
    chunk = 32
    n_chunks = rows_w // chunk
    nbuf = 4
    mesh = plsc.VectorSubcoreMesh(core_axis_name="c", subcore_axis_name="s")
    body = functools.partial(
        _sc_copy, rows_w=rows_w, chunk=chunk, n_chunks=n_chunks, nbuf=nbuf)
    k = pl.kernel(
        body,
        out_type=jax.ShapeDtypeStruct((S, D), W.dtype),
        mesh=mesh,
        scratch_types=[
            pltpu.VMEM((nbuf, chunk, D), W.dtype),
            pltpu.SemaphoreType.DMA((nbuf,)),
            pltpu.SemaphoreType.DMA((nbuf,)),
        ],
    )
    return k(W)
